# baseline (speedup 1.0000x reference)
import functools

import jax
import jax.numpy as jnp
from jax import lax
from jax.experimental import pallas as pl
from jax.experimental.pallas import tpu as pltpu
from jax.experimental.pallas import tpu_sc as plsc

_N = 10000
_E = 320000
_G = 64
_NSC = 2
_NSUB = 16
_NW = _NSC * _NSUB
_EPW = _E // _NW
_WIN = 80
_NWIN = _EPW // _WIN
_RPS = 632
_RPS_LAST = _N - 15 * _RPS
_BN = 400
_NB = _N // _BN
_F32 = jnp.float32


def _mesh():
    return plsc.VectorSubcoreMesh(core_axis_name="c", subcore_axis_name="s")


@functools.lru_cache(maxsize=None)
def _agg_kernel(nchunks):
    out_type = tuple(
        jax.ShapeDtypeStruct((2 * _N, 128), _F32) for _ in range(nchunks))
    scratch = (
        pltpu.VMEM_SHARED((_N, 128), _F32),
        pltpu.VMEM((_EPW,), jnp.int32),
        pltpu.VMEM((_NWIN, _WIN), jnp.int32),
        pltpu.VMEM((_WIN, 128), _F32),
        pltpu.VMEM((_WIN, 128), _F32),
        pltpu.SemaphoreType.DMA,
        pltpu.SemaphoreType.DMA,
        pltpu.SemaphoreType.DMA,
        pltpu.SemaphoreType.DMA,
    )

    def body(*refs):
        y_refs = refs[:nchunks]
        src_hbm, dst_hbm, zeros_hbm = refs[nchunks:nchunks + 3]
        p_refs = refs[nchunks + 3:2 * nchunks + 3]
        acc, sidx, didx, rows0, rows1, sg0, sg1, ss0, ss1 = \
            refs[2 * nchunks + 3:]
        cid = lax.axis_index("c")
        sid = lax.axis_index("s")
        w = cid * _NSUB + sid

        def on_own_rows(fn):
            @pl.when(sid < _NSUB - 1)
            def _():
                fn(pl.multiple_of(sid * _RPS, 8), _RPS)

            @pl.when(sid == _NSUB - 1)
            def _():
                fn((_NSUB - 1) * _RPS, _RPS_LAST)

        pltpu.sync_copy(
            src_hbm.at[pl.ds(pl.multiple_of(w * _EPW, 8), _EPW)], sidx)
        pltpu.sync_copy(dst_hbm.at[w], didx)

        def swin(i):
            return sidx.at[pl.ds(pl.multiple_of(i * _WIN, 8), _WIN)]

        for c in range(nchunks):
            y = y_refs[c]

            def half(i, rows, ss, rows_o, sg, sg_o, ss_o, y=y):
                pltpu.make_async_copy(y.at[swin(i)], rows, sg).wait()
                pltpu.async_copy(rows, acc.at[didx.at[i]], ss, add=True)

                @pl.when(i >= 1)
                def _():
                    pltpu.make_async_copy(rows_o, acc.at[didx.at[0]],
                                          ss_o).wait()

                @pl.when(i + 1 < _NWIN)
                def _():
                    pltpu.async_copy(y.at[swin(i + 1)], rows_o, sg_o)

            on_own_rows(lambda r0, nr: pltpu.sync_copy(
                zeros_hbm.at[pl.ds(r0, nr)], acc.at[pl.ds(r0, nr)]))
            plsc.subcore_barrier()
            pltpu.async_copy(y.at[swin(0)], rows0, sg0)

            def win_pair(k, carry):
                half(2 * k, rows0, ss0, rows1, sg0, sg1, ss1)
                half(2 * k + 1, rows1, ss1, rows0, sg1, sg0, ss0)
                return carry

            lax.fori_loop(0, _NWIN // 2, win_pair, 0)
            half(_NWIN - 1, rows0, ss0, rows1, sg0, sg1, ss1)
            pltpu.make_async_copy(rows0, acc.at[didx.at[0]], ss0).wait()
            plsc.subcore_barrier()
            p_ref = p_refs[c]
            on_own_rows(lambda r0, nr, p_ref=p_ref: pltpu.sync_copy(
                acc.at[pl.ds(r0, nr)],
                p_ref.at[pl.ds(pl.multiple_of(cid * _N + r0, 8), nr)]))

    return pl.kernel(body, out_type=out_type, mesh=_mesh(),
                     scratch_types=scratch)


@functools.lru_cache(maxsize=None)
def _deg_kernel():
    out_type = jax.ShapeDtypeStruct((2 * _N, 128), _F32)
    scratch = (
        pltpu.VMEM_SHARED((_N, 128), _F32),
        pltpu.VMEM((_NWIN, _WIN), jnp.int32),
        pltpu.VMEM((_WIN, 128), _F32),
        pltpu.SemaphoreType.DMA,
    )

    def body(dst_hbm, zeros_hbm, ones_hbm, out_ref, accd, didx, ones, sem):
        cid = lax.axis_index("c")
        sid = lax.axis_index("s")
        w = cid * _NSUB + sid
        pltpu.sync_copy(dst_hbm.at[w], didx)
        pltpu.sync_copy(ones_hbm, ones)

        def on_own_rows(fn):
            @pl.when(sid < _NSUB - 1)
            def _():
                fn(pl.multiple_of(sid * _RPS, 8), _RPS)

            @pl.when(sid == _NSUB - 1)
            def _():
                fn((_NSUB - 1) * _RPS, _RPS_LAST)

        on_own_rows(lambda r0, nr: pltpu.sync_copy(
            zeros_hbm.at[pl.ds(r0, nr)], accd.at[pl.ds(r0, nr)]))
        plsc.subcore_barrier()

        def scat(i, carry):
            pltpu.async_copy(ones, accd.at[didx.at[i]], sem, add=True)
            pltpu.make_async_copy(ones, accd.at[didx.at[i]], sem).wait()
            return carry

        lax.fori_loop(0, _NWIN, scat, 0)
        plsc.subcore_barrier()
        on_own_rows(lambda r0, nr: pltpu.sync_copy(
            accd.at[pl.ds(r0, nr)],
            out_ref.at[pl.ds(pl.multiple_of(cid * _N + r0, 8), nr)]))

    return pl.kernel(body, out_type=out_type, mesh=_mesh(),
                     scratch_types=scratch)


def _prep_call(x, deg2):
    def body(x_ref, d0, d1, dinv_ref, y_ref):
        deg = 1.0 + d0[:, :1] + d1[:, :1]
        dv = lax.rsqrt(deg)
        dinv_ref[...] = dv
        y_ref[...] = x_ref[...] * dv

    return pl.pallas_call(
        body,
        grid=(_NB,),
        in_specs=[
            pl.BlockSpec((_BN, 128), lambda i: (i, 0)),
            pl.BlockSpec((_BN, 128), lambda i: (i, 0)),
            pl.BlockSpec((_BN, 128), lambda i: (i + _NB, 0)),
        ],
        out_specs=[
            pl.BlockSpec((_BN, 1), lambda i: (i, 0)),
            pl.BlockSpec((_BN, 128), lambda i: (i, 0)),
        ],
        out_shape=[
            jax.ShapeDtypeStruct((_N, 1), _F32),
            jax.ShapeDtypeStruct((_N, 128), _F32),
        ],
    )(x, deg2, deg2)


@functools.lru_cache(maxsize=None)
def _layer_call(cin, cout, last):
    di = cin * 128
    do = cout * 128

    def body(*refs):
        dinv_ref = refs[0]
        y_refs = refs[1:1 + cin]
        p_refs = refs[1 + cin:1 + 3 * cin]
        w_ref = refs[1 + 3 * cin]
        b_ref = refs[2 + 3 * cin]
        k = 3 + 3 * cin
        wc_ref = refs[k] if last else None
        outs = refs[k + (1 if last else 0):]
        dv = dinv_ref[...]
        acc = None
        for c in range(cin):
            z = dv * (p_refs[2 * c][...] + p_refs[2 * c + 1][...]
                      + y_refs[c][...])
            part = jnp.dot(z, w_ref[c * 128:(c + 1) * 128, :],
                           preferred_element_type=_F32)
            acc = part if acc is None else acc + part
        h = jnp.maximum(acc + b_ref[...], 0.0)
        if last:
            wc = wc_ref[:1024, :] + wc_ref[1024:, :]
            outs[0][...] = jnp.dot(h, wc, preferred_element_type=_F32)
        else:
            hv = h * dv
            for c in range(cout):
                outs[c][...] = hv[:, c * 128:(c + 1) * 128]

    in_specs = [pl.BlockSpec((_BN, 1), lambda i: (i, 0))]
    in_specs += [pl.BlockSpec((_BN, 128), lambda i: (i, 0))
                 for _ in range(cin)]
    for _ in range(cin):
        in_specs.append(pl.BlockSpec((_BN, 128), lambda i: (i, 0)))
        in_specs.append(pl.BlockSpec((_BN, 128), lambda i: (i + _NB, 0)))
    in_specs.append(pl.BlockSpec((di, do), lambda i: (0, 0)))
    in_specs.append(pl.BlockSpec((1, do), lambda i: (0, 0)))
    if last:
        in_specs.append(pl.BlockSpec((2048, 1), lambda i: (0, 0)))
        out_specs = [pl.BlockSpec((_BN, 1), lambda i: (i, 0))]
        out_shape = [jax.ShapeDtypeStruct((_N, 1), _F32)]
    else:
        out_specs = [pl.BlockSpec((_BN, 128), lambda i: (i, 0))
                     for _ in range(cout)]
        out_shape = [jax.ShapeDtypeStruct((_N, 128), _F32)
                     for _ in range(cout)]
    return pl.pallas_call(body, grid=(_NB,), in_specs=in_specs,
                          out_specs=out_specs, out_shape=out_shape)


def _pool_call(t_row, batch_row, bc2):
    def body(t_ref, b_ref, bc_ref, out_ref):
        gids = lax.broadcasted_iota(jnp.int32, (_G, _N), 0)
        mask = b_ref[...] == gids
        tb = jnp.broadcast_to(t_ref[...], (_G, _N))
        ssum = jnp.sum(jnp.where(mask, tb, 0.0), axis=1, keepdims=True)
        scnt = jnp.sum(mask.astype(_F32), axis=1, keepdims=True)
        out_ref[...] = ssum / jnp.maximum(scnt, 1.0) + bc_ref[...]

    return pl.pallas_call(
        body,
        out_shape=jax.ShapeDtypeStruct((_G, 1), _F32),
    )(t_row, batch_row, bc2)


def kernel(x, edge_index, batch, W1, b1, W2, b2, W3, b3, W4, b4, Wc, bc):
    src = edge_index[0].astype(jnp.int32)
    dst = edge_index[1].astype(jnp.int32).reshape(_NW, _NWIN, _WIN)
    zeros128 = jnp.zeros((_N, 128), _F32)
    ones80 = jnp.ones((_WIN, 128), _F32)

    deg2 = _deg_kernel()(dst, zeros128, ones80)
    dinv, y0 = _prep_call(x, deg2)

    ys = [y0]
    t = None
    layers = ((W1, b1, 1, 1), (W2, b2, 1, 2), (W3, b3, 2, 4), (W4, b4, 4, 8))
    for li, (W, b, cin, cout) in enumerate(layers):
        ps = _agg_kernel(cin)(*ys, src, dst, zeros128)
        if not isinstance(ps, (tuple, list)):
            ps = (ps,)
        last = li == 3
        args = [dinv] + list(ys)
        for c in range(cin):
            args += [ps[c], ps[c]]
        args += [W, b.reshape(1, -1)]
        if last:
            args.append(Wc)
        outs = _layer_call(cin, cout, last)(*args)
        if last:
            t = outs[0]
        else:
            ys = list(outs)

    return _pool_call(t.reshape(1, _N),
                      batch.astype(jnp.int32).reshape(1, _N),
                      bc.reshape(1, 1))


# speedup vs baseline: 1.2943x; 1.2943x over previous
import functools

import jax
import jax.numpy as jnp
from jax import lax
from jax.experimental import pallas as pl
from jax.experimental.pallas import tpu as pltpu
from jax.experimental.pallas import tpu_sc as plsc

_N = 10000
_E = 320000
_G = 64
_NSC = 2
_NSUB = 16
_NW = _NSC * _NSUB
_EPW = _E // _NW
_WIN = 80
_NWIN = _EPW // _WIN
_RPS = 632
_RPS_LAST = _N - 15 * _RPS
_BN = 400
_NB = _N // _BN
_F32 = jnp.float32


def _mesh():
    return plsc.VectorSubcoreMesh(core_axis_name="c", subcore_axis_name="s")


@functools.lru_cache(maxsize=None)
def _agg_kernel(nchunks):
    out_type = tuple(
        jax.ShapeDtypeStruct((2 * _N, 128), _F32) for _ in range(nchunks))
    nbuf = 3
    nidx = 6
    scratch = (
        pltpu.VMEM_SHARED((_N, 128), _F32),
        pltpu.VMEM((_NWIN, _WIN), jnp.int32),
    ) + tuple(pltpu.VMEM((_WIN, 128), _F32) for _ in range(nbuf)) \
      + tuple(pltpu.VMEM((_WIN,), jnp.int32) for _ in range(nidx)) \
      + tuple(pltpu.SemaphoreType.DMA for _ in range(2 * nbuf + nidx))

    def body(*refs):
        y_refs = refs[:nchunks]
        src_hbm, dst_hbm, zeros_hbm = refs[nchunks:nchunks + 3]
        p_refs = refs[nchunks + 3:2 * nchunks + 3]
        rest = refs[2 * nchunks + 3:]
        acc, didx = rest[:2]
        rows = rest[2:2 + nbuf]
        sidx = rest[2 + nbuf:2 + nbuf + nidx]
        sg = rest[2 + nbuf + nidx:2 + 2 * nbuf + nidx]
        ss = rest[2 + 2 * nbuf + nidx:2 + 3 * nbuf + nidx]
        si = rest[2 + 3 * nbuf + nidx:]
        cid = lax.axis_index("c")
        sid = lax.axis_index("s")
        w = cid * _NSUB + sid
        ebase = pl.multiple_of(w * _EPW, 8)

        def on_own_rows(fn):
            @pl.when(sid < _NSUB - 1)
            def _():
                fn(pl.multiple_of(sid * _RPS, 8), _RPS)

            @pl.when(sid == _NSUB - 1)
            def _():
                fn((_NSUB - 1) * _RPS, _RPS_LAST)

        pltpu.sync_copy(dst_hbm.at[w], didx)

        def idx_load(i, j):
            pltpu.async_copy(
                src_hbm.at[pl.ds(pl.multiple_of(ebase + i * _WIN, 8), _WIN)],
                sidx[j], si[j])

        def idx_wait(i, j):
            pltpu.make_async_copy(
                src_hbm.at[pl.ds(ebase, _WIN)], sidx[j], si[j]).wait()

        for c in range(nchunks):
            y = y_refs[c]

            def process(i, b, j, issue_next=True, y=y):
                pltpu.make_async_copy(y.at[sidx[j]], rows[b], sg[b]).wait()
                pltpu.async_copy(rows[b], acc.at[didx.at[i]], ss[b], add=True)
                pltpu.make_async_copy(rows[b], acc.at[didx.at[i]],
                                      ss[b]).wait()
                if issue_next:
                    @pl.when(i + nbuf < _NWIN)
                    def _():
                        jn = (j + nbuf) % nidx
                        idx_wait(i + nbuf, jn)
                        pltpu.async_copy(y.at[sidx[jn]], rows[b], sg[b])

                    @pl.when(i + nidx < _NWIN)
                    def _():
                        idx_load(i + nidx, j)

            on_own_rows(lambda r0, nr: pltpu.sync_copy(
                zeros_hbm.at[pl.ds(r0, nr)], acc.at[pl.ds(r0, nr)]))
            for j in range(nbuf):
                idx_load(j, j)
            plsc.subcore_barrier()
            for b in range(nbuf):
                idx_wait(b, b)
                pltpu.async_copy(y.at[sidx[b]], rows[b], sg[b])
                idx_load(b + nbuf, (b + nbuf) % nidx)

            def win_six(k, carry):
                i0 = nidx * k
                for m in range(nidx):
                    process(i0 + m, m % nbuf, m)
                return carry

            lax.fori_loop(0, _NWIN // nidx, win_six, 0)
            for t in range(_NWIN - _NWIN % nidx, _NWIN):
                process(t, t % nbuf, t % nidx)
            plsc.subcore_barrier()
            p_ref = p_refs[c]
            on_own_rows(lambda r0, nr, p_ref=p_ref: pltpu.sync_copy(
                acc.at[pl.ds(r0, nr)],
                p_ref.at[pl.ds(pl.multiple_of(cid * _N + r0, 8), nr)]))

    return pl.kernel(body, out_type=out_type, mesh=_mesh(),
                     scratch_types=scratch)


@functools.lru_cache(maxsize=None)
def _deg_kernel():
    out_type = jax.ShapeDtypeStruct((2 * _N, 128), _F32)
    scratch = (
        pltpu.VMEM_SHARED((_N, 128), _F32),
        pltpu.VMEM((_NWIN, _WIN), jnp.int32),
        pltpu.VMEM((_WIN, 128), _F32),
        pltpu.SemaphoreType.DMA,
    )

    def body(dst_hbm, zeros_hbm, ones_hbm, out_ref, accd, didx, ones, sem):
        cid = lax.axis_index("c")
        sid = lax.axis_index("s")
        w = cid * _NSUB + sid
        pltpu.sync_copy(dst_hbm.at[w], didx)
        pltpu.sync_copy(ones_hbm, ones)

        def on_own_rows(fn):
            @pl.when(sid < _NSUB - 1)
            def _():
                fn(pl.multiple_of(sid * _RPS, 8), _RPS)

            @pl.when(sid == _NSUB - 1)
            def _():
                fn((_NSUB - 1) * _RPS, _RPS_LAST)

        on_own_rows(lambda r0, nr: pltpu.sync_copy(
            zeros_hbm.at[pl.ds(r0, nr)], accd.at[pl.ds(r0, nr)]))
        plsc.subcore_barrier()

        def scat(i, carry):
            pltpu.async_copy(ones, accd.at[didx.at[i]], sem, add=True)
            pltpu.make_async_copy(ones, accd.at[didx.at[i]], sem).wait()
            return carry

        lax.fori_loop(0, _NWIN, scat, 0)
        plsc.subcore_barrier()
        on_own_rows(lambda r0, nr: pltpu.sync_copy(
            accd.at[pl.ds(r0, nr)],
            out_ref.at[pl.ds(pl.multiple_of(cid * _N + r0, 8), nr)]))

    return pl.kernel(body, out_type=out_type, mesh=_mesh(),
                     scratch_types=scratch)


def _rd(a):
    return a.astype(jnp.bfloat16).astype(_F32)


def _prep_call(x, deg2):
    def body(x_ref, d0, d1, dinv_ref, y_ref):
        deg = 1.0 + d0[:, :1] + d1[:, :1]
        dv = lax.rsqrt(deg)
        dinv_ref[...] = dv
        y_ref[...] = _rd(x_ref[...]) * dv

    return pl.pallas_call(
        body,
        grid=(_NB,),
        in_specs=[
            pl.BlockSpec((_BN, 128), lambda i: (i, 0)),
            pl.BlockSpec((_BN, 128), lambda i: (i, 0)),
            pl.BlockSpec((_BN, 128), lambda i: (i + _NB, 0)),
        ],
        out_specs=[
            pl.BlockSpec((_BN, 1), lambda i: (i, 0)),
            pl.BlockSpec((_BN, 128), lambda i: (i, 0)),
        ],
        out_shape=[
            jax.ShapeDtypeStruct((_N, 1), _F32),
            jax.ShapeDtypeStruct((_N, 128), _F32),
        ],
    )(x, deg2, deg2)


@functools.lru_cache(maxsize=None)
def _layer_call(cin, cout, last):
    di = cin * 128
    do = cout * 128

    def body(*refs):
        dinv_ref = refs[0]
        y_refs = refs[1:1 + cin]
        p_refs = refs[1 + cin:1 + 3 * cin]
        w_ref = refs[1 + 3 * cin]
        b_ref = refs[2 + 3 * cin]
        outs = refs[3 + 3 * cin:]
        dv = dinv_ref[...]
        acc = None
        for c in range(cin):
            z = dv * (p_refs[2 * c][...] + p_refs[2 * c + 1][...]
                      + y_refs[c][...])
            part = jnp.dot(z, _rd(w_ref[c * 128:(c + 1) * 128, :]),
                           precision=lax.Precision.HIGHEST,
                           preferred_element_type=_F32)
            acc = part if acc is None else acc + part
        h = jnp.maximum(acc + b_ref[...], 0.0)
        hv = h if last else _rd(h) * dv
        for c in range(cout):
            outs[c][...] = hv[:, c * 128:(c + 1) * 128]

    in_specs = [pl.BlockSpec((_BN, 1), lambda i: (i, 0))]
    in_specs += [pl.BlockSpec((_BN, 128), lambda i: (i, 0))
                 for _ in range(cin)]
    for _ in range(cin):
        in_specs.append(pl.BlockSpec((_BN, 128), lambda i: (i, 0)))
        in_specs.append(pl.BlockSpec((_BN, 128), lambda i: (i + _NB, 0)))
    in_specs.append(pl.BlockSpec((di, do), lambda i: (0, 0)))
    in_specs.append(pl.BlockSpec((1, do), lambda i: (0, 0)))
    out_specs = [pl.BlockSpec((_BN, 128), lambda i: (i, 0))
                 for _ in range(cout)]
    out_shape = [jax.ShapeDtypeStruct((_N, 128), _F32)
                 for _ in range(cout)]
    return pl.pallas_call(body, grid=(_NB,), in_specs=in_specs,
                          out_specs=out_specs, out_shape=out_shape)


def _pool_call(h_chunks, batch_row, Wc, bc2):
    nch = len(h_chunks)

    def body(*refs):
        h_refs = refs[:nch]
        b_ref, wc_ref, bc_ref, out_ref, ssum, scnt = refs[nch:]
        i = pl.program_id(0)

        @pl.when(i == 0)
        def _():
            ssum[...] = jnp.zeros((_G, 8 * 128), _F32)
            scnt[...] = jnp.zeros((_G, 1), _F32)

        h_blk = jnp.concatenate([hr[...] for hr in h_refs], axis=1)
        gids = lax.broadcasted_iota(jnp.int32, (_BN, _G), 1)
        mask = (b_ref[...] == gids).astype(_F32)
        dn = (((0,), (0,)), ((), ()))
        ssum[...] += lax.dot_general(mask, h_blk, dn,
                                     precision=lax.Precision.HIGHEST,
                                     preferred_element_type=_F32)
        scnt[...] += lax.dot_general(mask, jnp.ones((_BN, 1), _F32), dn,
                                     precision=lax.Precision.HIGHEST,
                                     preferred_element_type=_F32)

        @pl.when(i == _NB - 1)
        def _():
            m = ssum[...] / jnp.maximum(scnt[...], 1.0)
            wcs = _rd(wc_ref[:1024, :]) + _rd(wc_ref[1024:, :])
            out_ref[...] = jnp.dot(_rd(m), wcs,
                                   precision=lax.Precision.HIGHEST,
                                   preferred_element_type=_F32) + bc_ref[...]

    return pl.pallas_call(
        body,
        grid=(_NB,),
        in_specs=[pl.BlockSpec((_BN, 128), lambda i: (i, 0))
                  for _ in range(nch)]
        + [
            pl.BlockSpec((_BN, 1), lambda i: (i, 0)),
            pl.BlockSpec((2048, 1), lambda i: (0, 0)),
            pl.BlockSpec((1, 1), lambda i: (0, 0)),
        ],
        out_specs=pl.BlockSpec((_G, 1), lambda i: (0, 0)),
        out_shape=jax.ShapeDtypeStruct((_G, 1), _F32),
        scratch_shapes=[pltpu.VMEM((_G, 8 * 128), _F32),
                        pltpu.VMEM((_G, 1), _F32)],
    )(*h_chunks, batch_row, Wc, bc2)


def kernel(x, edge_index, batch, W1, b1, W2, b2, W3, b3, W4, b4, Wc, bc):
    src = edge_index[0].astype(jnp.int32)
    dst = edge_index[1].astype(jnp.int32).reshape(_NW, _NWIN, _WIN)
    zeros128 = jnp.zeros((_N, 128), _F32)
    ones80 = jnp.ones((_WIN, 128), _F32)

    deg2 = _deg_kernel()(dst, zeros128, ones80)
    dinv, y0 = _prep_call(x, deg2)

    ys = [y0]
    h4 = None
    layers = ((W1, b1, 1, 1), (W2, b2, 1, 2), (W3, b3, 2, 4), (W4, b4, 4, 8))
    for li, (W, b, cin, cout) in enumerate(layers):
        ps = _agg_kernel(cin)(*ys, src, dst, zeros128)
        if not isinstance(ps, (tuple, list)):
            ps = (ps,)
        last = li == 3
        args = [dinv] + list(ys)
        for c in range(cin):
            args += [ps[c], ps[c]]
        args += [W, b.reshape(1, -1)]
        outs = _layer_call(cin, cout, last)(*args)
        if last:
            h4 = list(outs)
        else:
            ys = list(outs)

    return _pool_call(h4,
                      batch.astype(jnp.int32).reshape(_N, 1),
                      Wc,
                      bc.reshape(1, 1))
